# baseline (speedup 1.0000x reference)
import jax
import jax.numpy as jnp
from jax import lax
from jax.experimental import pallas as pl
from jax.experimental.pallas import tpu as pltpu

B, C, H, W = 2, 1024, 16, 16
E = 8
ND = 6
HF = int(C * 4.0)
N_TOK = B * H * W
HF_B = 512
N_HFB = HF // HF_B

TM = 64
NT = 24
S = NT * TM

_EPS64 = 2.220446049250313e-16
_HIGH = lax.Precision.HIGHEST


def _routing(xf, prompt, de_cls, w_g, gate_boost, degra_W, degra_b):
    w1g = w_g[:C, :]
    w2g = w_g[C:, :]
    pbias = lax.dot_general(prompt, w2g, (((1,), (0,)), ((), ())),
                            preferred_element_type=jnp.float32)
    dbias = lax.dot_general(de_cls, degra_W, (((1,), (1,)), ((), ())),
                            preferred_element_type=jnp.float32)
    bias_b = pbias + gate_boost * (dbias + degra_b)
    logits = lax.dot_general(xf, w1g, (((1,), (0,)), ((), ())),
                             preferred_element_type=jnp.float32)
    row = lax.broadcasted_iota(jnp.int32, (N_TOK, E), 0)
    per_tok_bias = jnp.where(row < (N_TOK // B), bias_b[0:1, :], bias_b[1:2, :])
    logits = logits + per_tok_bias

    neg = jnp.float32(-jnp.inf)
    m1 = jnp.full((N_TOK, 1), neg, dtype=jnp.float32)
    m2 = jnp.full((N_TOK, 1), neg, dtype=jnp.float32)
    a1 = jnp.zeros((N_TOK, 1), dtype=jnp.int32)
    a2 = jnp.zeros((N_TOK, 1), dtype=jnp.int32)
    for j in range(E):
        lj = logits[:, j:j + 1]
        jn = jnp.int32(j)
        new1 = lj > m1
        new2 = jnp.logical_and(jnp.logical_not(new1), lj > m2)
        m2 = jnp.where(new1, m1, jnp.where(new2, lj, m2))
        a2 = jnp.where(new1, a1, jnp.where(new2, jn, a2))
        m1 = jnp.where(new1, lj, m1)
        a1 = jnp.where(new1, jn, a1)
    u = jnp.exp(m2 - m1)
    denom = 1.0 + u
    g1 = 1.0 / denom
    g2 = u / denom
    return a1, a2, g1, g2


def _lane_balance(v):
    m = jnp.sum(v) / E
    var = jnp.sum((v - m) ** 2) / (E - 1)
    return var / (m * m + 1e-10)


def _dispatch_kernel(xf_ref, prompt_ref, de_cls_ref, w_g_ref, boost_ref,
                     degW_ref, degb_ref,
                     xs_ref, rt_ref, gs_ref, meta_ref, loss_ref):
    xf = xf_ref[...]
    a1, a2, g1, g2 = _routing(
        xf, prompt_ref[...], de_cls_ref[...], w_g_ref[...],
        boost_ref[0, 0], degW_ref[...], degb_ref[0, :])

    e_lane = lax.broadcasted_iota(jnp.int32, (N_TOK, E), 1)
    oh1 = (a1 == e_lane).astype(jnp.float32)
    oh2 = (a2 == e_lane).astype(jnp.float32)

    wv = jnp.sum(oh1 * g1 + oh2 * g2, axis=0, keepdims=True)
    sv = jnp.sum(oh1 + oh2 * (g2 > 0.0).astype(jnp.float32),
                 axis=0, keepdims=True)
    loss = _lane_balance(wv) + _lane_balance(sv)
    loss_ref[...] = jnp.reshape(loss, (1, 1))

    ii = lax.broadcasted_iota(jnp.int32, (N_TOK, N_TOK), 0)
    jj = lax.broadcasted_iota(jnp.int32, (N_TOK, N_TOK), 1)
    lower = (jj <= ii).astype(jnp.float32)
    cum1 = lax.dot_general(lower, oh1, (((1,), (0,)), ((), ())),
                           preferred_element_type=jnp.float32)
    cum2 = lax.dot_general(lower, oh2, (((1,), (0,)), ((), ())),
                           preferred_element_type=jnp.float32)
    cnt1 = jnp.sum(oh1, axis=0, keepdims=True)
    cnt = cnt1 + jnp.sum(oh2, axis=0, keepdims=True)

    ntiles = jnp.floor((cnt + (TM - 1.0)) * (1.0 / TM))
    ee1 = lax.broadcasted_iota(jnp.int32, (E, E), 0)
    ee2 = lax.broadcasted_iota(jnp.int32, (E, E), 1)
    incl = (ee1 <= ee2).astype(jnp.float32)
    cume = lax.dot_general(ntiles, incl, (((1,), (0,)), ((), ())),
                           preferred_element_type=jnp.float32)
    ts_excl = cume - ntiles
    start_slot = ts_excl * float(TM)
    ntu = cume[:, E - 1:E]
    lane_e = lax.broadcasted_iota(jnp.int32, (1, E), 1).astype(jnp.float32)
    last_e = jnp.max(lane_e * (ntiles > 0.0).astype(jnp.float32))

    slot1 = jnp.zeros((N_TOK, 1), jnp.float32)
    slot2 = jnp.zeros((N_TOK, 1), jnp.float32)
    for e in range(E):
        st = start_slot[:, e:e + 1]
        c1 = cnt1[:, e:e + 1]
        slot1 = slot1 + oh1[:, e:e + 1] * (st + cum1[:, e:e + 1])
        slot2 = slot2 + oh2[:, e:e + 1] * (st + c1 + cum2[:, e:e + 1])
    slot1 = slot1 - 1.0
    slot2 = slot2 - 1.0

    s_lane = lax.broadcasted_iota(jnp.int32, (N_TOK, S), 1).astype(jnp.float32)
    ohs1 = (slot1 == s_lane).astype(jnp.float32)
    ohs2 = (slot2 == s_lane).astype(jnp.float32)
    ohs = ohs1 + ohs2

    xs_ref[...] = lax.dot_general(ohs, xf, (((0,), (0,)), ((), ())),
                                  preferred_element_type=jnp.float32)
    n_col = lax.broadcasted_iota(jnp.int32, (N_TOK, 1), 0).astype(jnp.float32)
    rt = lax.dot_general(ohs, n_col, (((0,), (0,)), ((), ())),
                         precision=_HIGH,
                         preferred_element_type=jnp.float32)
    rt_ref[...] = rt.astype(jnp.int32)
    gs = (lax.dot_general(ohs1, g1, (((0,), (0,)), ((), ())),
                          precision=_HIGH, preferred_element_type=jnp.float32)
          + lax.dot_general(ohs2, g2, (((0,), (0,)), ((), ())),
                            precision=_HIGH,
                            preferred_element_type=jnp.float32))
    gs_ref[...] = gs

    t_lane = lax.broadcasted_iota(jnp.int32, (1, 128), 1).astype(jnp.float32)
    eot = jnp.zeros((1, 128), jnp.float32)
    for e in range(E):
        eot = eot + (t_lane >= cume[:, e:e + 1]).astype(jnp.float32)
    eot = jnp.where(t_lane < ntu, eot, last_e)
    meta = jnp.where(t_lane == float(NT), ntu, eot)
    meta_ref[...] = meta.astype(jnp.int32)


def _ffn_kernel(meta_sref, xs_ref, gs_ref, rt_ref,
                w1_ref, b1_ref, w2_ref, b2_ref,
                y_ref, acc_s):
    h = pl.program_id(0)
    t = pl.program_id(1)
    ntu = meta_sref[NT]
    valid = t < ntu
    row0 = t * TM

    @pl.when(valid)
    def _tile_ffn():
        xblk = xs_ref[pl.ds(row0, TM), :]
        w1b = w1_ref[0]
        hid = lax.dot_general(xblk, w1b, (((1,), (1,)), ((), ())),
                              preferred_element_type=jnp.float32)
        hid = hid + b1_ref[0]
        hid = 0.5 * hid * (1.0 + lax.erf(hid * jnp.float32(0.7071067811865476)))
        w2b = w2_ref[0]
        contrib = lax.dot_general(hid, w2b, (((1,), (1,)), ((), ())),
                                  preferred_element_type=jnp.float32)

        @pl.when(h == 0)
        def _():
            acc_s[pl.ds(row0, TM), :] = contrib

        @pl.when(h != 0)
        def _():
            acc_s[pl.ds(row0, TM), :] = acc_s[pl.ds(row0, TM), :] + contrib

    @pl.when(h == N_HFB - 1)
    def _finalize_tile():
        outv = acc_s[pl.ds(row0, TM), :] + b2_ref[0]
        gex = gs_ref[pl.ds(row0, TM), :] * jnp.exp(outv)
        acc_s[pl.ds(row0, TM), :] = jnp.where(valid, gex,
                                              jnp.zeros_like(gex))

        @pl.when(t == NT - 1)
        def _combine():
            rt = rt_ref[...]
            n_lane = lax.broadcasted_iota(jnp.int32, (S, N_TOK), 1)
            ohc = (rt == n_lane).astype(jnp.float32)
            ens = lax.dot_general(ohc, acc_s[...], (((0,), (0,)), ((), ())),
                                  precision=_HIGH,
                                  preferred_element_type=jnp.float32)
            ens = jnp.where(ens == 0.0, jnp.float32(_EPS64), ens)
            y_ref[...] = jnp.log(ens)


def kernel(x, prompt, de_cls, w_g, gate_boost, degra_W, degra_b,
           W1, b1, W2, b2):
    b, c, h, w_ = x.shape
    xf = jnp.transpose(x, (0, 2, 3, 1)).reshape(-1, c)
    boost = jnp.reshape(gate_boost, (1, 1)).astype(jnp.float32)
    degb = jnp.reshape(degra_b, (1, E))

    xs, rt, gs, meta, loss = pl.pallas_call(
        _dispatch_kernel,
        grid=(1,),
        in_specs=[
            pl.BlockSpec((N_TOK, C), lambda i: (0, 0)),
            pl.BlockSpec((B, C), lambda i: (0, 0)),
            pl.BlockSpec((B, ND), lambda i: (0, 0)),
            pl.BlockSpec((2 * C, E), lambda i: (0, 0)),
            pl.BlockSpec((1, 1), lambda i: (0, 0)),
            pl.BlockSpec((E, ND), lambda i: (0, 0)),
            pl.BlockSpec((1, E), lambda i: (0, 0)),
        ],
        out_specs=[
            pl.BlockSpec((S, C), lambda i: (0, 0)),
            pl.BlockSpec((S, 1), lambda i: (0, 0)),
            pl.BlockSpec((S, 1), lambda i: (0, 0)),
            pl.BlockSpec((1, 128), lambda i: (0, 0)),
            pl.BlockSpec((1, 1), lambda i: (0, 0)),
        ],
        out_shape=[
            jax.ShapeDtypeStruct((S, C), jnp.float32),
            jax.ShapeDtypeStruct((S, 1), jnp.int32),
            jax.ShapeDtypeStruct((S, 1), jnp.float32),
            jax.ShapeDtypeStruct((1, 128), jnp.int32),
            jax.ShapeDtypeStruct((1, 1), jnp.float32),
        ],
    )(xf, prompt, de_cls, w_g, boost, degra_W, degb)

    grid_spec = pltpu.PrefetchScalarGridSpec(
        num_scalar_prefetch=1,
        grid=(N_HFB, NT),
        in_specs=[
            pl.BlockSpec((S, C), lambda hh, t, s: (0, 0)),
            pl.BlockSpec((S, 1), lambda hh, t, s: (0, 0)),
            pl.BlockSpec((S, 1), lambda hh, t, s: (0, 0)),
            pl.BlockSpec((1, HF_B, C), lambda hh, t, s: (s[t], hh, 0)),
            pl.BlockSpec((1, 1, HF_B), lambda hh, t, s: (s[t], 0, hh)),
            pl.BlockSpec((1, C, HF_B), lambda hh, t, s: (s[t], 0, hh)),
            pl.BlockSpec((1, 1, C), lambda hh, t, s: (s[t], 0, 0)),
        ],
        out_specs=[
            pl.BlockSpec((N_TOK, C), lambda hh, t, s: (0, 0)),
        ],
        scratch_shapes=[pltpu.VMEM((S, C), jnp.float32)],
    )
    (y,) = pl.pallas_call(
        _ffn_kernel,
        grid_spec=grid_spec,
        out_shape=[jax.ShapeDtypeStruct((N_TOK, C), jnp.float32)],
        compiler_params=pltpu.CompilerParams(
            dimension_semantics=("arbitrary", "arbitrary"),
        ),
    )(meta.reshape(128), xs, gs, rt,
      W1, b1.reshape(E, 1, HF), W2, b2.reshape(E, 1, C))

    y = y.reshape(b, h, w_, c).transpose(0, 3, 1, 2)
    return y, jnp.reshape(loss, ())


# speedup vs baseline: 2.2405x; 2.2405x over previous
import jax
import jax.numpy as jnp
from jax import lax
from jax.experimental import pallas as pl
from jax.experimental.pallas import tpu as pltpu

B, C, H, W = 2, 1024, 16, 16
E = 8
ND = 6
HF = int(C * 4.0)
N_TOK = B * H * W
HF_B = 1024
N_HFB = HF // HF_B

_EPS64 = 2.220446049250313e-16


def _routing(xf, prompt, de_cls, w_g, gate_boost, degra_W, degra_b):
    w1g = w_g[:C, :]
    w2g = w_g[C:, :]
    pbias = lax.dot_general(prompt, w2g, (((1,), (0,)), ((), ())),
                            preferred_element_type=jnp.float32)
    dbias = lax.dot_general(de_cls, degra_W, (((1,), (1,)), ((), ())),
                            preferred_element_type=jnp.float32)
    bias_b = pbias + gate_boost * (dbias + degra_b)
    logits = lax.dot_general(xf, w1g, (((1,), (0,)), ((), ())),
                             preferred_element_type=jnp.float32)
    row = lax.broadcasted_iota(jnp.int32, (N_TOK, E), 0)
    per_tok_bias = jnp.where(row < (N_TOK // B), bias_b[0:1, :], bias_b[1:2, :])
    logits = logits + per_tok_bias

    neg = jnp.float32(-jnp.inf)
    m1 = jnp.full((N_TOK, 1), neg, dtype=jnp.float32)
    m2 = jnp.full((N_TOK, 1), neg, dtype=jnp.float32)
    a1 = jnp.zeros((N_TOK, 1), dtype=jnp.int32)
    a2 = jnp.zeros((N_TOK, 1), dtype=jnp.int32)
    for j in range(E):
        lj = logits[:, j:j + 1]
        jn = jnp.int32(j)
        new1 = lj > m1
        new2 = jnp.logical_and(jnp.logical_not(new1), lj > m2)
        m2 = jnp.where(new1, m1, jnp.where(new2, lj, m2))
        a2 = jnp.where(new1, a1, jnp.where(new2, jn, a2))
        m1 = jnp.where(new1, lj, m1)
        a1 = jnp.where(new1, jn, a1)
    u = jnp.exp(m2 - m1)
    denom = 1.0 + u
    g1 = 1.0 / denom
    g2 = u / denom
    return a1, a2, g1, g2


def _balance_terms(vals):
    n = len(vals)
    s = vals[0]
    for v in vals[1:]:
        s = s + v
    m = s / n
    sq = (vals[0] - m) ** 2
    for v in vals[1:]:
        sq = sq + (v - m) ** 2
    var = sq / (n - 1)
    return var / (m * m + 1e-10)


def _kernel(xf_ref, xb_ref, prompt_ref, de_cls_ref, w_g_ref, boost_ref,
            degW_ref, degb_ref, w1_ref, b1_ref, w2_ref, b2_ref,
            y_ref, loss_ref,
            a1_s, a2_s, g1_s, g2_s, outacc_s, ensacc_s):
    e = pl.program_id(0)
    h = pl.program_id(1)

    @pl.when(jnp.logical_and(e == 0, h == 0))
    def _do_routing():
        a1, a2, g1, g2 = _routing(
            xf_ref[...], prompt_ref[...], de_cls_ref[...], w_g_ref[...],
            boost_ref[0, 0], degW_ref[...], degb_ref[0, :])
        a1_s[...] = a1
        a2_s[...] = a2
        g1_s[...] = g1
        g2_s[...] = g2
        wv, sv = [], []
        for ee in range(E):
            ge = (jnp.where(a1 == ee, g1, 0.0) + jnp.where(a2 == ee, g2, 0.0))
            wv.append(jnp.sum(ge))
            sv.append(jnp.sum((ge > 0.0).astype(jnp.float32)))
        loss = _balance_terms(wv) + _balance_terms(sv)
        loss_ref[...] = jnp.reshape(loss, (1, 1))

    xb = xf_ref[...]
    w1b = w1_ref[0]
    hid = lax.dot_general(xb, w1b, (((1,), (1,)), ((), ())),
                          preferred_element_type=jnp.float32)
    hid = hid + b1_ref[0]
    hid = 0.5 * hid * (1.0 + lax.erf(hid * jnp.float32(0.7071067811865476)))
    w2b = w2_ref[0]
    contrib = lax.dot_general(hid, w2b,
                              (((1,), (1,)), ((), ())),
                              preferred_element_type=jnp.float32)

    @pl.when(h == 0)
    def _init_out():
        outacc_s[...] = contrib

    @pl.when(h != 0)
    def _acc_out():
        outacc_s[...] = outacc_s[...] + contrib

    @pl.when(h == N_HFB - 1)
    def _combine():
        out = outacc_s[...] + b2_ref[0]
        gate = (jnp.where(a1_s[...] == e, g1_s[...], 0.0)
                + jnp.where(a2_s[...] == e, g2_s[...], 0.0))
        term = gate * jnp.exp(out)

        @pl.when(e == 0)
        def _():
            ensacc_s[...] = term

        @pl.when(e != 0)
        def _():
            ensacc_s[...] = ensacc_s[...] + term

        @pl.when(e == E - 1)
        def _final():
            ens = ensacc_s[...]
            ens = jnp.where(ens == 0.0, jnp.float32(_EPS64), ens)
            y_ref[...] = jnp.log(ens)


def kernel(x, prompt, de_cls, w_g, gate_boost, degra_W, degra_b,
           W1, b1, W2, b2):
    b, c, h, w_ = x.shape
    xf = jnp.transpose(x, (0, 2, 3, 1)).reshape(-1, c)
    boost = jnp.reshape(gate_boost, (1, 1)).astype(jnp.float32)
    degb = jnp.reshape(degra_b, (1, E))

    grid = (E, N_HFB)
    y, loss = pl.pallas_call(
        _kernel,
        grid=grid,
        in_specs=[
            pl.BlockSpec((N_TOK, C), lambda e, hh: (0, 0)),
            pl.BlockSpec((N_TOK, C), lambda e, hh: (0, 0)),
            pl.BlockSpec((B, C), lambda e, hh: (0, 0)),
            pl.BlockSpec((B, ND), lambda e, hh: (0, 0)),
            pl.BlockSpec((2 * C, E), lambda e, hh: (0, 0)),
            pl.BlockSpec((1, 1), lambda e, hh: (0, 0)),
            pl.BlockSpec((E, ND), lambda e, hh: (0, 0)),
            pl.BlockSpec((1, E), lambda e, hh: (0, 0)),
            pl.BlockSpec((1, HF_B, C), lambda e, hh: (e, hh, 0)),
            pl.BlockSpec((1, 1, HF_B), lambda e, hh: (e, 0, hh)),
            pl.BlockSpec((1, C, HF_B), lambda e, hh: (e, 0, hh)),
            pl.BlockSpec((1, 1, C), lambda e, hh: (e, 0, 0)),
        ],
        out_specs=[
            pl.BlockSpec((N_TOK, C), lambda e, hh: (0, 0)),
            pl.BlockSpec((1, 1), lambda e, hh: (0, 0)),
        ],
        out_shape=[
            jax.ShapeDtypeStruct((N_TOK, C), jnp.float32),
            jax.ShapeDtypeStruct((1, 1), jnp.float32),
        ],
        scratch_shapes=[
            pltpu.VMEM((N_TOK, 1), jnp.int32),
            pltpu.VMEM((N_TOK, 1), jnp.int32),
            pltpu.VMEM((N_TOK, 1), jnp.float32),
            pltpu.VMEM((N_TOK, 1), jnp.float32),
            pltpu.VMEM((N_TOK, C), jnp.float32),
            pltpu.VMEM((N_TOK, C), jnp.float32),
        ],
        compiler_params=pltpu.CompilerParams(
            dimension_semantics=("arbitrary", "arbitrary"),
        ),
    )(xf, xf.astype(jnp.bfloat16), prompt, de_cls, w_g, boost, degra_W, degb,
      W1, b1.reshape(E, 1, HF), W2, b2.reshape(E, 1, C))

    y = y.reshape(b, h, w_, c).transpose(0, 3, 1, 2)
    return y, jnp.reshape(loss, ())


# speedup vs baseline: 2.3948x; 1.0689x over previous
import jax
import jax.numpy as jnp
from jax import lax
from jax.experimental import pallas as pl
from jax.experimental.pallas import tpu as pltpu

B, C, H, W = 2, 1024, 16, 16
E = 8
ND = 6
HF = int(C * 4.0)
N_TOK = B * H * W
HF_B = 2048
N_HFB = HF // HF_B

_EPS64 = 2.220446049250313e-16


def _routing(xf, prompt, de_cls, w_g, gate_boost, degra_W, degra_b):
    w1g = w_g[:C, :]
    w2g = w_g[C:, :]
    pbias = lax.dot_general(prompt, w2g, (((1,), (0,)), ((), ())),
                            preferred_element_type=jnp.float32)
    dbias = lax.dot_general(de_cls, degra_W, (((1,), (1,)), ((), ())),
                            preferred_element_type=jnp.float32)
    bias_b = pbias + gate_boost * (dbias + degra_b)
    logits = lax.dot_general(xf, w1g, (((1,), (0,)), ((), ())),
                             preferred_element_type=jnp.float32)
    row = lax.broadcasted_iota(jnp.int32, (N_TOK, E), 0)
    per_tok_bias = jnp.where(row < (N_TOK // B), bias_b[0:1, :], bias_b[1:2, :])
    logits = logits + per_tok_bias

    neg = jnp.float32(-jnp.inf)
    m1 = jnp.full((N_TOK, 1), neg, dtype=jnp.float32)
    m2 = jnp.full((N_TOK, 1), neg, dtype=jnp.float32)
    a1 = jnp.zeros((N_TOK, 1), dtype=jnp.int32)
    a2 = jnp.zeros((N_TOK, 1), dtype=jnp.int32)
    for j in range(E):
        lj = logits[:, j:j + 1]
        jn = jnp.int32(j)
        new1 = lj > m1
        new2 = jnp.logical_and(jnp.logical_not(new1), lj > m2)
        m2 = jnp.where(new1, m1, jnp.where(new2, lj, m2))
        a2 = jnp.where(new1, a1, jnp.where(new2, jn, a2))
        m1 = jnp.where(new1, lj, m1)
        a1 = jnp.where(new1, jn, a1)
    u = jnp.exp(m2 - m1)
    denom = 1.0 + u
    g1 = 1.0 / denom
    g2 = u / denom
    return a1, a2, g1, g2


def _balance_terms(vals):
    n = len(vals)
    s = vals[0]
    for v in vals[1:]:
        s = s + v
    m = s / n
    sq = (vals[0] - m) ** 2
    for v in vals[1:]:
        sq = sq + (v - m) ** 2
    var = sq / (n - 1)
    return var / (m * m + 1e-10)


def _kernel(xf_ref, xb_ref, prompt_ref, de_cls_ref, w_g_ref, boost_ref,
            degW_ref, degb_ref, w1_ref, b1_ref, w2_ref, b2_ref,
            y_ref, loss_ref,
            a1_s, a2_s, g1_s, g2_s, outacc_s, ensacc_s):
    e = pl.program_id(0)
    h = pl.program_id(1)

    @pl.when(jnp.logical_and(e == 0, h == 0))
    def _do_routing():
        a1, a2, g1, g2 = _routing(
            xf_ref[...], prompt_ref[...], de_cls_ref[...], w_g_ref[...],
            boost_ref[0, 0], degW_ref[...], degb_ref[0, :])
        a1_s[...] = a1
        a2_s[...] = a2
        g1_s[...] = g1
        g2_s[...] = g2
        wv, sv = [], []
        for ee in range(E):
            ge = (jnp.where(a1 == ee, g1, 0.0) + jnp.where(a2 == ee, g2, 0.0))
            wv.append(jnp.sum(ge))
            sv.append(jnp.sum((ge > 0.0).astype(jnp.float32)))
        loss = _balance_terms(wv) + _balance_terms(sv)
        loss_ref[...] = jnp.reshape(loss, (1, 1))

    xb = xf_ref[...]
    w1b = w1_ref[0]
    hid = lax.dot_general(xb, w1b, (((1,), (1,)), ((), ())),
                          preferred_element_type=jnp.float32)
    hid = hid + b1_ref[0]
    hid = 0.5 * hid * (1.0 + lax.erf(hid * jnp.float32(0.7071067811865476)))
    w2b = w2_ref[0]
    contrib = lax.dot_general(hid, w2b,
                              (((1,), (1,)), ((), ())),
                              preferred_element_type=jnp.float32)

    @pl.when(h == 0)
    def _init_out():
        outacc_s[...] = contrib

    @pl.when(h != 0)
    def _acc_out():
        outacc_s[...] = outacc_s[...] + contrib

    @pl.when(h == N_HFB - 1)
    def _combine():
        out = outacc_s[...] + b2_ref[0]
        gate = (jnp.where(a1_s[...] == e, g1_s[...], 0.0)
                + jnp.where(a2_s[...] == e, g2_s[...], 0.0))
        term = gate * jnp.exp(out)

        @pl.when(e == 0)
        def _():
            ensacc_s[...] = term

        @pl.when(e != 0)
        def _():
            ensacc_s[...] = ensacc_s[...] + term

        @pl.when(e == E - 1)
        def _final():
            ens = ensacc_s[...]
            ens = jnp.where(ens == 0.0, jnp.float32(_EPS64), ens)
            y_ref[...] = jnp.log(ens)


def kernel(x, prompt, de_cls, w_g, gate_boost, degra_W, degra_b,
           W1, b1, W2, b2):
    b, c, h, w_ = x.shape
    xf = jnp.transpose(x, (0, 2, 3, 1)).reshape(-1, c)
    boost = jnp.reshape(gate_boost, (1, 1)).astype(jnp.float32)
    degb = jnp.reshape(degra_b, (1, E))

    grid = (E, N_HFB)
    y, loss = pl.pallas_call(
        _kernel,
        grid=grid,
        in_specs=[
            pl.BlockSpec((N_TOK, C), lambda e, hh: (0, 0)),
            pl.BlockSpec((N_TOK, C), lambda e, hh: (0, 0)),
            pl.BlockSpec((B, C), lambda e, hh: (0, 0)),
            pl.BlockSpec((B, ND), lambda e, hh: (0, 0)),
            pl.BlockSpec((2 * C, E), lambda e, hh: (0, 0)),
            pl.BlockSpec((1, 1), lambda e, hh: (0, 0)),
            pl.BlockSpec((E, ND), lambda e, hh: (0, 0)),
            pl.BlockSpec((1, E), lambda e, hh: (0, 0)),
            pl.BlockSpec((1, HF_B, C), lambda e, hh: (e, hh, 0)),
            pl.BlockSpec((1, 1, HF_B), lambda e, hh: (e, 0, hh)),
            pl.BlockSpec((1, C, HF_B), lambda e, hh: (e, 0, hh)),
            pl.BlockSpec((1, 1, C), lambda e, hh: (e, 0, 0)),
        ],
        out_specs=[
            pl.BlockSpec((N_TOK, C), lambda e, hh: (0, 0)),
            pl.BlockSpec((1, 1), lambda e, hh: (0, 0)),
        ],
        out_shape=[
            jax.ShapeDtypeStruct((N_TOK, C), jnp.float32),
            jax.ShapeDtypeStruct((1, 1), jnp.float32),
        ],
        scratch_shapes=[
            pltpu.VMEM((N_TOK, 1), jnp.int32),
            pltpu.VMEM((N_TOK, 1), jnp.int32),
            pltpu.VMEM((N_TOK, 1), jnp.float32),
            pltpu.VMEM((N_TOK, 1), jnp.float32),
            pltpu.VMEM((N_TOK, C), jnp.float32),
            pltpu.VMEM((N_TOK, C), jnp.float32),
        ],
        compiler_params=pltpu.CompilerParams(
            dimension_semantics=("arbitrary", "arbitrary"),
        ),
    )(xf, xf.astype(jnp.bfloat16), prompt, de_cls, w_g, boost, degra_W, degb,
      W1, b1.reshape(E, 1, HF), W2, b2.reshape(E, 1, C))

    y = y.reshape(b, h, w_, c).transpose(0, 3, 1, 2)
    return y, jnp.reshape(loss, ())
